# baseline (speedup 1.0000x reference)
import jax
import jax.numpy as jnp
from jax.experimental import pallas as pl
from jax.experimental.pallas import tpu as pltpu

_N, _C, _J = 65536, 80, 600
_RB = 1024


def _body(idx_ref, x_ref, o_ref):
    iota_c = jax.lax.broadcasted_iota(jnp.int32, (_C, _J), 0)
    onehot = (idx_ref[0, :][None, :] == iota_c).astype(jnp.float32)
    o_ref[...] = jnp.dot(x_ref[...], onehot, preferred_element_type=jnp.float32)


def kernel(o_conf, obj2hoi):
    idx = obj2hoi.astype(jnp.int32).reshape(1, _J)
    return pl.pallas_call(
        _body,
        grid=(_N // _RB,),
        in_specs=[
            pl.BlockSpec((1, _J), lambda i: (0, 0)),
            pl.BlockSpec((_RB, _C), lambda i: (i, 0)),
        ],
        out_specs=pl.BlockSpec((_RB, _J), lambda i: (i, 0)),
        out_shape=jax.ShapeDtypeStruct((_N, _J), jnp.float32),
    )(idx, o_conf)
